# Initial kernel scaffold; baseline (speedup 1.0000x reference)
import jax
import jax.numpy as jnp
from jax.experimental import pallas as pl


def kernel(X, gradient, hessian):
    raise NotImplementedError("write your pallas kernel here")


# speedup vs baseline: 78.7648x; 78.7648x over previous
import functools

import jax
import jax.numpy as jnp
from jax import lax
from jax.experimental import pallas as pl
from jax.experimental.pallas import tpu as pltpu
from jax.experimental.pallas import tpu_sc as plsc

MAXB = 256
NFEAT = 64
NC, NS, LANES = 2, 16, 16
NW = NC * NS
HIST = NFEAT * MAXB


def _sc_partial_hists(X, gradient, hessian):
    N = X.shape[0]
    per_w = N // NW
    CH = 512
    n_ch = per_w // CH
    mesh = plsc.VectorSubcoreMesh(
        core_axis_name="c", subcore_axis_name="s", num_cores=NC, num_subcores=NS
    )

    @functools.partial(
        pl.kernel,
        out_type=(
            jax.ShapeDtypeStruct((NW, HIST), jnp.float32),
            jax.ShapeDtypeStruct((NW, HIST), jnp.float32),
        ),
        mesh=mesh,
        compiler_params=pltpu.CompilerParams(needs_layout_passes=False),
        scratch_types=[
            pltpu.VMEM((CH, NFEAT), jnp.int32),
            pltpu.VMEM((CH,), jnp.float32),
            pltpu.VMEM((CH,), jnp.float32),
            pltpu.VMEM((HIST,), jnp.float32),
            pltpu.VMEM((HIST,), jnp.float32),
        ],
    )
    def hist_kernel(x_hbm, g_hbm, h_hbm, og_hbm, oh_hbm, xv, gv, hv, hg, hh):
        wid = lax.axis_index("s") * NC + lax.axis_index("c")
        base = wid * per_w
        zeros = jnp.zeros((LANES,), jnp.float32)

        @pl.loop(0, HIST // LANES)
        def _zero(j):
            hg[pl.ds(j * LANES, LANES)] = zeros
            hh[pl.ds(j * LANES, LANES)] = zeros

        offs = [
            (lax.iota(jnp.int32, LANES) + t * LANES) * MAXB
            for t in range(NFEAT // LANES)
        ]

        @pl.loop(0, n_ch)
        def _chunk(c):
            start = base + c * CH
            pltpu.sync_copy(x_hbm.at[pl.ds(start, CH)], xv)
            pltpu.sync_copy(g_hbm.at[pl.ds(start, CH)], gv)
            pltpu.sync_copy(h_hbm.at[pl.ds(start, CH)], hv)

            @pl.loop(0, CH // LANES)
            def _blk(b):
                i0 = b * LANES
                gblk = gv[pl.ds(i0, LANES)]
                hblk = hv[pl.ds(i0, LANES)]
                for j in range(LANES):
                    gvec = jnp.full((LANES,), gblk[j], jnp.float32)
                    hvec = jnp.full((LANES,), hblk[j], jnp.float32)
                    for t in range(NFEAT // LANES):
                        idx = xv[i0 + j, pl.ds(t * LANES, LANES)] + offs[t]
                        plsc.addupdate_scatter(hg, [idx], gvec)
                        plsc.addupdate_scatter(hh, [idx], hvec)

        pltpu.sync_copy(hg, og_hbm.at[wid])
        pltpu.sync_copy(hh, oh_hbm.at[wid])

    return hist_kernel(X, gradient, hessian)


def _tc_merge_suffix(pg, ph):
    def body(pg_ref, ph_ref, og_ref, oh_ref):
        sg = jnp.sum(pg_ref[...], axis=0)
        sh = jnp.sum(ph_ref[...], axis=0)
        row = lax.broadcasted_iota(jnp.int32, (MAXB, MAXB), 0)
        col = lax.broadcasted_iota(jnp.int32, (MAXB, MAXB), 1)
        tri = (row > col).astype(jnp.float32)
        og_ref[...] = jnp.dot(sg, tri, preferred_element_type=jnp.float32)
        oh_ref[...] = jnp.dot(sh, tri, preferred_element_type=jnp.float32)

    return pl.pallas_call(
        body,
        out_shape=(
            jax.ShapeDtypeStruct((NFEAT, MAXB), jnp.float32),
            jax.ShapeDtypeStruct((NFEAT, MAXB), jnp.float32),
        ),
    )(pg, ph)


def kernel(X, gradient, hessian):
    pg, ph = _sc_partial_hists(X, gradient, hessian)
    pg = pg.reshape(NW, NFEAT, MAXB)
    ph = ph.reshape(NW, NFEAT, MAXB)
    Gl, Hl = _tc_merge_suffix(pg, ph)
    return (Gl[None], Hl[None])


# speedup vs baseline: 99.7741x; 1.2667x over previous
import functools

import jax
import jax.numpy as jnp
from jax import lax
from jax.experimental import pallas as pl
from jax.experimental.pallas import tpu as pltpu
from jax.experimental.pallas import tpu_sc as plsc

MAXB = 256
NFEAT = 64
NC, NS, LANES = 2, 16, 16
NW = NC * NS
HIST = NFEAT * MAXB


def _sc_partial_hists(X, gradient, hessian):
    N = X.shape[0]
    per_w = N // NW
    CH = 256
    n_ch = per_w // CH
    mesh = plsc.VectorSubcoreMesh(
        core_axis_name="c", subcore_axis_name="s", num_cores=NC, num_subcores=NS
    )

    @functools.partial(
        pl.kernel,
        out_type=(
            jax.ShapeDtypeStruct((NW, HIST), jnp.float32),
            jax.ShapeDtypeStruct((NW, HIST), jnp.float32),
        ),
        mesh=mesh,
        compiler_params=pltpu.CompilerParams(needs_layout_passes=False),
        scratch_types=[
            pltpu.VMEM((2, CH, NFEAT), jnp.int32),
            pltpu.VMEM((per_w,), jnp.float32),
            pltpu.VMEM((per_w,), jnp.float32),
            pltpu.VMEM((HIST,), jnp.float32),
            pltpu.VMEM((HIST,), jnp.float32),
            pltpu.SemaphoreType.DMA,
            pltpu.SemaphoreType.DMA,
            pltpu.SemaphoreType.DMA,
        ],
    )
    def hist_kernel(x_hbm, g_hbm, h_hbm, og_hbm, oh_hbm, xv, gv, hv, hg, hh,
                    sem0, sem1, semw):
        wid = lax.axis_index("s") * NC + lax.axis_index("c")
        base = wid * per_w
        sems = (sem0, sem1)

        wg = pltpu.async_copy(g_hbm.at[pl.ds(base, per_w)], gv, semw)
        wh = pltpu.async_copy(h_hbm.at[pl.ds(base, per_w)], hv, semw)
        pltpu.async_copy(x_hbm.at[pl.ds(base, CH)], xv.at[0], sem0)
        pltpu.async_copy(x_hbm.at[pl.ds(base + CH, CH)], xv.at[1], sem1)

        zeros = jnp.zeros((LANES,), jnp.float32)

        @pl.loop(0, HIST // LANES)
        def _zero(j):
            hg[pl.ds(j * LANES, LANES)] = zeros
            hh[pl.ds(j * LANES, LANES)] = zeros

        offs = [
            lax.iota(jnp.int32, LANES) + t * LANES
            for t in range(NFEAT // LANES)
        ]
        wg.wait()
        wh.wait()

        @pl.loop(0, n_ch, step=2)
        def _chunk(c):
            for b in range(2):
                cc = c + b
                pltpu.make_async_copy(
                    x_hbm.at[pl.ds(0, CH)], xv.at[b], sems[b]
                ).wait()

                @pl.loop(0, CH // LANES)
                def _blk(blk):
                    i0 = blk * LANES
                    gblk = gv[pl.ds(cc * CH + i0, LANES)]
                    hblk = hv[pl.ds(cc * CH + i0, LANES)]
                    for j in range(LANES):
                        gvec = jnp.full((LANES,), gblk[j], jnp.float32)
                        hvec = jnp.full((LANES,), hblk[j], jnp.float32)
                        for t in range(NFEAT // LANES):
                            xvec = xv[b, i0 + j, pl.ds(t * LANES, LANES)]
                            idx = xvec * NFEAT + offs[t]
                            plsc.addupdate_scatter(hg, [idx], gvec)
                            plsc.addupdate_scatter(hh, [idx], hvec)

                @pl.when(cc + 2 < n_ch)
                def _prefetch():
                    s = base + (cc + 2) * CH
                    pltpu.async_copy(x_hbm.at[pl.ds(s, CH)], xv.at[b], sems[b])

        pltpu.sync_copy(hg, og_hbm.at[wid])
        pltpu.sync_copy(hh, oh_hbm.at[wid])

    return hist_kernel(X, gradient, hessian)


def _tc_merge_suffix(pg, ph):
    def body(pg_ref, ph_ref, og_ref, oh_ref):
        sg = jnp.sum(pg_ref[...], axis=0)
        sh = jnp.sum(ph_ref[...], axis=0)
        row = lax.broadcasted_iota(jnp.int32, (MAXB, MAXB), 0)
        col = lax.broadcasted_iota(jnp.int32, (MAXB, MAXB), 1)
        tri = (row > col).astype(jnp.float32)
        dn = (((0,), (0,)), ((), ()))
        og_ref[...] = lax.dot_general(sg, tri, dn,
                                      preferred_element_type=jnp.float32)
        oh_ref[...] = lax.dot_general(sh, tri, dn,
                                      preferred_element_type=jnp.float32)

    return pl.pallas_call(
        body,
        out_shape=(
            jax.ShapeDtypeStruct((NFEAT, MAXB), jnp.float32),
            jax.ShapeDtypeStruct((NFEAT, MAXB), jnp.float32),
        ),
    )(pg, ph)


def kernel(X, gradient, hessian):
    pg, ph = _sc_partial_hists(X, gradient, hessian)
    pg = pg.reshape(NW, MAXB, NFEAT)
    ph = ph.reshape(NW, MAXB, NFEAT)
    Gl, Hl = _tc_merge_suffix(pg, ph)
    return (Gl[None], Hl[None])


# speedup vs baseline: 148.7160x; 1.4905x over previous
import functools

import jax
import jax.numpy as jnp
from jax import lax
from jax.experimental import pallas as pl
from jax.experimental.pallas import tpu as pltpu
from jax.experimental.pallas import tpu_sc as plsc

MAXB = 256
NFEAT = 64
NC, NS, LANES = 2, 16, 16
NW = NC * NS
HIST = NFEAT * MAXB


def _sc_partial_hists(X, gradient, hessian):
    N = X.shape[0]
    per_w = N // NW
    CH = 256
    n_ch = per_w // CH
    mesh = plsc.VectorSubcoreMesh(
        core_axis_name="c", subcore_axis_name="s", num_cores=NC, num_subcores=NS
    )

    @functools.partial(
        pl.kernel,
        out_type=(
            jax.ShapeDtypeStruct((NW, HIST), jnp.float32),
            jax.ShapeDtypeStruct((NW, HIST), jnp.float32),
        ),
        mesh=mesh,
        compiler_params=pltpu.CompilerParams(needs_layout_passes=False),
        scratch_types=[
            pltpu.VMEM((2, CH, NFEAT), jnp.int32),
            pltpu.VMEM((per_w,), jnp.float32),
            pltpu.VMEM((per_w,), jnp.float32),
            pltpu.VMEM((HIST,), jnp.float32),
            pltpu.VMEM((HIST,), jnp.float32),
            pltpu.SemaphoreType.DMA,
            pltpu.SemaphoreType.DMA,
            pltpu.SemaphoreType.DMA,
        ],
    )
    def hist_kernel(x_hbm, g_hbm, h_hbm, og_hbm, oh_hbm, xv, gv, hv, hg, hh,
                    sem0, sem1, semw):
        wid = lax.axis_index("s") * NC + lax.axis_index("c")
        base = wid * per_w
        sems = (sem0, sem1)

        wg = pltpu.async_copy(g_hbm.at[pl.ds(base, per_w)], gv, semw)
        wh = pltpu.async_copy(h_hbm.at[pl.ds(base, per_w)], hv, semw)
        pltpu.async_copy(x_hbm.at[pl.ds(base, CH)], xv.at[0], sem0)
        pltpu.async_copy(x_hbm.at[pl.ds(base + CH, CH)], xv.at[1], sem1)

        zeros = jnp.zeros((LANES,), jnp.float32)

        @pl.loop(0, HIST // LANES)
        def _zero(j):
            hg[pl.ds(j * LANES, LANES)] = zeros
            hh[pl.ds(j * LANES, LANES)] = zeros

        offs = [
            lax.iota(jnp.int32, LANES) + t * LANES
            for t in range(NFEAT // LANES)
        ]
        wg.wait()
        wh.wait()

        @pl.loop(0, n_ch, step=2)
        def _chunk(c):
            for b in range(2):
                cc = c + b
                pltpu.make_async_copy(
                    x_hbm.at[pl.ds(0, CH)], xv.at[b], sems[b]
                ).wait()

                @plsc.parallel_loop(0, CH // LANES, unroll=2)
                def _blk(blk):
                    i0 = blk * LANES
                    gblk = gv[pl.ds(cc * CH + i0, LANES)]
                    hblk = hv[pl.ds(cc * CH + i0, LANES)]
                    for j in range(LANES):
                        gvec = jnp.full((LANES,), gblk[j], jnp.float32)
                        hvec = jnp.full((LANES,), hblk[j], jnp.float32)
                        for t in range(NFEAT // LANES):
                            xvec = xv[b, i0 + j, pl.ds(t * LANES, LANES)]
                            idx = xvec * NFEAT + offs[t]
                            plsc.addupdate_scatter(hg, [idx], gvec)
                            plsc.addupdate_scatter(hh, [idx], hvec)

                @pl.when(cc + 2 < n_ch)
                def _prefetch():
                    s = base + (cc + 2) * CH
                    pltpu.async_copy(x_hbm.at[pl.ds(s, CH)], xv.at[b], sems[b])

        pltpu.sync_copy(hg, og_hbm.at[wid])
        pltpu.sync_copy(hh, oh_hbm.at[wid])

    return hist_kernel(X, gradient, hessian)


def _tc_merge_suffix(pg, ph):
    def body(pg_ref, ph_ref, og_ref, oh_ref):
        sg = jnp.sum(pg_ref[...], axis=0)
        sh = jnp.sum(ph_ref[...], axis=0)
        row = lax.broadcasted_iota(jnp.int32, (MAXB, MAXB), 0)
        col = lax.broadcasted_iota(jnp.int32, (MAXB, MAXB), 1)
        tri = (row > col).astype(jnp.float32)
        dn = (((0,), (0,)), ((), ()))
        og_ref[...] = lax.dot_general(sg, tri, dn,
                                      preferred_element_type=jnp.float32)
        oh_ref[...] = lax.dot_general(sh, tri, dn,
                                      preferred_element_type=jnp.float32)

    return pl.pallas_call(
        body,
        out_shape=(
            jax.ShapeDtypeStruct((NFEAT, MAXB), jnp.float32),
            jax.ShapeDtypeStruct((NFEAT, MAXB), jnp.float32),
        ),
    )(pg, ph)


def kernel(X, gradient, hessian):
    pg, ph = _sc_partial_hists(X, gradient, hessian)
    pg = pg.reshape(NW, MAXB, NFEAT)
    ph = ph.reshape(NW, MAXB, NFEAT)
    Gl, Hl = _tc_merge_suffix(pg, ph)
    return (Gl[None], Hl[None])
